# speedup vs baseline: 1.7833x; 1.7833x over previous
import functools

import jax
import jax.numpy as jnp
from jax import lax
from jax.experimental import pallas as pl
from jax.experimental.pallas import tpu as pltpu

_EPS = 1e-5
_NEG = float(jnp.finfo(jnp.float32).min)


def _shift_lanes(a, k, fill):
    if k == 0:
        return a
    c, l = a.shape
    f = jnp.full((c, abs(k)), fill, a.dtype)
    if k > 0:
        return jnp.concatenate([a[:, k:], f], axis=1)
    return jnp.concatenate([f, a[:, :l + k]], axis=1)


def _wshift(a, dw, w, fill):
    s = _shift_lanes(a, dw, fill)
    if dw == 0:
        return s
    wi = lax.rem(lax.broadcasted_iota(jnp.int32, a.shape, 1), jnp.int32(w))
    if dw > 0:
        valid = wi < (w - dw)
    else:
        valid = wi >= (-dw)
    return jnp.where(valid, s, fill)


def _accum_stats(st_ref, vals, pid):
    s = jnp.sum(vals, axis=1, keepdims=True)
    q = jnp.sum(vals * vals, axis=1, keepdims=True)
    lanes = lax.broadcasted_iota(jnp.int32, st_ref.shape, 1)
    local = jnp.where(lanes < 1, s, q)

    @pl.when(pid == 0)
    def _():
        st_ref[...] = local

    @pl.when(pid > 0)
    def _():
        st_ref[...] = st_ref[...] + local


def _s1_body(x_ref, wx_ref, wp_ref, keep_ref, mid_ref, st_ref, *, w, out1):
    xx = x_ref[...]
    y = jnp.dot(wx_ref[...], xx.astype(jnp.bfloat16),
                preferred_element_type=jnp.float32)

    rowm = jnp.maximum(xx, jnp.maximum(_wshift(xx, 1, w, _NEG),
                                       _wshift(xx, -1, w, _NEG)))
    pooled = jnp.maximum(rowm, jnp.maximum(_shift_lanes(rowm, w, _NEG),
                                           _shift_lanes(rowm, -w, _NEG)))
    yp = jnp.dot(wp_ref[...], pooled.astype(jnp.bfloat16),
                 preferred_element_type=jnp.float32)

    keep = jnp.concatenate([y[:out1], yp], axis=0)
    mid = y[out1:]
    keep_ref[...] = keep
    mid_ref[...] = mid
    _accum_stats(st_ref, jnp.concatenate([keep, mid], axis=0),
                 pl.program_id(0))


def _s2_body(mid_ref, sc_ref, sh_ref, w3_ref, w5_ref, acc2_ref, st_ref, *,
             w, red3):
    mid = mid_ref[...]
    a = jnp.maximum(mid * sc_ref[:, 0:1] + sh_ref[:, 0:1], 0.0)
    r3 = a[:red3]
    r5 = a[red3:]

    wv3 = {dw: _wshift(r3, dw, w, 0.0) for dw in (-1, 0, 1)}
    taps3 = [_shift_lanes(wv3[kw - 1], (kh - 1) * w, 0.0)
             for kh in range(3) for kw in range(3)]
    x3 = jnp.concatenate(taps3, axis=0).astype(jnp.bfloat16)
    y3 = jnp.dot(w3_ref[...], x3, preferred_element_type=jnp.float32)

    wv5 = {dw: _wshift(r5, dw, w, 0.0) for dw in (-2, -1, 0, 1, 2)}
    taps5 = [_shift_lanes(wv5[kw - 2], (kh - 2) * w, 0.0)
             for kh in range(5) for kw in range(5)]
    x5 = jnp.concatenate(taps5, axis=0).astype(jnp.bfloat16)
    y5 = jnp.dot(w5_ref[...], x5, preferred_element_type=jnp.float32)

    out = jnp.concatenate([y3, y5], axis=0)
    acc2_ref[...] = out
    _accum_stats(st_ref, out, pl.program_id(0))


def _s3_body(keep_ref, acc2_ref, sc_ref, sh_ref, out_ref, *, out1):
    k = keep_ref[...]
    raw = jnp.concatenate([k[:out1], acc2_ref[...], k[out1:]], axis=0)
    out_ref[...] = jnp.maximum(raw * sc_ref[:, 0:1] + sh_ref[:, 0:1], 0.0)


def _fold_bn(stats, gamma, beta, count):
    mean = stats[:, 0] / count
    var = stats[:, 1] / count - mean * mean
    scale = gamma * lax.rsqrt(var + _EPS)
    shift = beta - mean * scale
    return scale, shift


def _bcast(v):
    return jnp.broadcast_to(v.reshape(-1, 1), (v.shape[0], 128))


def kernel(x, b1_w, b1_gamma, b1_beta, b2a_w, b2a_gamma, b2a_beta,
           b2b_w, b2b_gamma, b2b_beta, b3a_w, b3a_gamma, b3a_beta,
           b3b_w, b3b_gamma, b3b_beta, b4_w, b4_gamma, b4_beta):
    x = x.astype(jnp.float32)
    n, cin, h, w = x.shape
    hw = h * w
    out1 = b1_w.shape[-1]
    red3, out3 = b2a_w.shape[-1], b2b_w.shape[-1]
    red5, out5 = b3a_w.shape[-1], b3b_w.shape[-1]
    outp = b4_w.shape[-1]
    cmid = red3 + red5
    ckeep = out1 + outp
    c2 = out3 + out5
    cout = out1 + out3 + out5 + outp

    x3 = x.reshape(n, cin, hw)
    wx = jnp.concatenate([b1_w.reshape(cin, out1),
                          b2a_w.reshape(cin, red3),
                          b3a_w.reshape(cin, red5)], axis=1)
    wx_t = wx.T.astype(jnp.bfloat16)
    wp_t = b4_w.reshape(cin, outp).T.astype(jnp.bfloat16)

    keep, mid, stats1 = pl.pallas_call(
        functools.partial(_s1_body, w=w, out1=out1),
        grid=(n,),
        in_specs=[
            pl.BlockSpec((None, cin, hw), lambda i: (i, 0, 0)),
            pl.BlockSpec((out1 + cmid, cin), lambda i: (0, 0)),
            pl.BlockSpec((outp, cin), lambda i: (0, 0)),
        ],
        out_specs=(
            pl.BlockSpec((None, ckeep, hw), lambda i: (i, 0, 0)),
            pl.BlockSpec((None, cmid, hw), lambda i: (i, 0, 0)),
            pl.BlockSpec((ckeep + cmid, 128), lambda i: (0, 0)),
        ),
        out_shape=(
            jax.ShapeDtypeStruct((n, ckeep, hw), jnp.float32),
            jax.ShapeDtypeStruct((n, cmid, hw), jnp.float32),
            jax.ShapeDtypeStruct((ckeep + cmid, 128), jnp.float32),
        ),
        compiler_params=pltpu.CompilerParams(
            dimension_semantics=("arbitrary",)),
    )(x3, wx_t, wp_t)

    count = float(n * hw)
    g_keep = jnp.concatenate([b1_gamma, b4_gamma])
    bt_keep = jnp.concatenate([b1_beta, b4_beta])
    g_mid = jnp.concatenate([b2a_gamma, b3a_gamma])
    bt_mid = jnp.concatenate([b2a_beta, b3a_beta])
    sc_k, sh_k = _fold_bn(stats1[:ckeep], g_keep, bt_keep, count)
    sc_m, sh_m = _fold_bn(stats1[ckeep:], g_mid, bt_mid, count)

    w3_t = b2b_w.reshape(9 * red3, out3).T.astype(jnp.bfloat16)
    w5_t = b3b_w.reshape(25 * red5, out5).T.astype(jnp.bfloat16)

    acc2, stats2 = pl.pallas_call(
        functools.partial(_s2_body, w=w, red3=red3),
        grid=(n,),
        in_specs=[
            pl.BlockSpec((None, cmid, hw), lambda i: (i, 0, 0)),
            pl.BlockSpec((cmid, 128), lambda i: (0, 0)),
            pl.BlockSpec((cmid, 128), lambda i: (0, 0)),
            pl.BlockSpec((out3, 9 * red3), lambda i: (0, 0)),
            pl.BlockSpec((out5, 25 * red5), lambda i: (0, 0)),
        ],
        out_specs=(
            pl.BlockSpec((None, c2, hw), lambda i: (i, 0, 0)),
            pl.BlockSpec((c2, 128), lambda i: (0, 0)),
        ),
        out_shape=(
            jax.ShapeDtypeStruct((n, c2, hw), jnp.float32),
            jax.ShapeDtypeStruct((c2, 128), jnp.float32),
        ),
        compiler_params=pltpu.CompilerParams(
            dimension_semantics=("arbitrary",)),
    )(mid, _bcast(sc_m), _bcast(sh_m), w3_t, w5_t)

    g2 = jnp.concatenate([b2b_gamma, b3b_gamma])
    bt2 = jnp.concatenate([b2b_beta, b3b_beta])
    sc_2, sh_2 = _fold_bn(stats2, g2, bt2, count)

    sc_all = jnp.concatenate([sc_k[:out1], sc_2, sc_k[out1:]])
    sh_all = jnp.concatenate([sh_k[:out1], sh_2, sh_k[out1:]])

    out = pl.pallas_call(
        functools.partial(_s3_body, out1=out1),
        grid=(n,),
        in_specs=[
            pl.BlockSpec((None, ckeep, hw), lambda i: (i, 0, 0)),
            pl.BlockSpec((None, c2, hw), lambda i: (i, 0, 0)),
            pl.BlockSpec((cout, 128), lambda i: (0, 0)),
            pl.BlockSpec((cout, 128), lambda i: (0, 0)),
        ],
        out_specs=pl.BlockSpec((None, cout, hw), lambda i: (i, 0, 0)),
        out_shape=jax.ShapeDtypeStruct((n, cout, hw), jnp.float32),
        compiler_params=pltpu.CompilerParams(
            dimension_semantics=("parallel",)),
    )(keep, acc2, _bcast(sc_all), _bcast(sh_all))

    return out.reshape(n, cout, h, w)


# speedup vs baseline: 1.8575x; 1.0416x over previous
import functools

import jax
import jax.numpy as jnp
from jax import lax
from jax.experimental import pallas as pl
from jax.experimental.pallas import tpu as pltpu

_EPS = 1e-5
_NEG = float(jnp.finfo(jnp.float32).min)


def _shift_lanes(a, k, fill):
    if k == 0:
        return a
    c, l = a.shape
    f = jnp.full((c, abs(k)), fill, a.dtype)
    if k > 0:
        return jnp.concatenate([a[:, k:], f], axis=1)
    return jnp.concatenate([f, a[:, :l + k]], axis=1)


def _wshift(a, dw, w, fill):
    s = _shift_lanes(a, dw, fill)
    if dw == 0:
        return s
    wi = lax.rem(lax.broadcasted_iota(jnp.int32, a.shape, 1), jnp.int32(w))
    if dw > 0:
        valid = wi < (w - dw)
    else:
        valid = wi >= (-dw)
    return jnp.where(valid, s, fill)


def _accum_stats(st_ref, vals, pid):
    s = jnp.sum(vals, axis=1, keepdims=True)
    q = jnp.sum(vals * vals, axis=1, keepdims=True)
    lanes = lax.broadcasted_iota(jnp.int32, st_ref.shape, 1)
    local = jnp.where(lanes < 1, s, q)

    @pl.when(pid == 0)
    def _():
        st_ref[...] = local

    @pl.when(pid > 0)
    def _():
        st_ref[...] = st_ref[...] + local


def _s1_body(x_ref, wx_ref, wp_ref, keep_ref, mid_ref, st_ref, *, w, out1):
    xx = x_ref[...]
    y = jnp.dot(wx_ref[...], xx.astype(jnp.bfloat16),
                preferred_element_type=jnp.float32)

    rowm = jnp.maximum(xx, jnp.maximum(_wshift(xx, 1, w, _NEG),
                                       _wshift(xx, -1, w, _NEG)))
    pooled = jnp.maximum(rowm, jnp.maximum(_shift_lanes(rowm, w, _NEG),
                                           _shift_lanes(rowm, -w, _NEG)))
    yp = jnp.dot(wp_ref[...], pooled.astype(jnp.bfloat16),
                 preferred_element_type=jnp.float32)

    keep = jnp.concatenate([y[:out1], yp], axis=0)
    mid = y[out1:]
    keep_ref[...] = keep.astype(jnp.bfloat16)
    mid_ref[...] = mid.astype(jnp.bfloat16)
    _accum_stats(st_ref, jnp.concatenate([keep, mid], axis=0),
                 pl.program_id(0))


def _s2_body(mid_ref, sc_ref, sh_ref, w3_ref, w5_ref, acc2_ref, st_ref, *,
             w, red3):
    mid = mid_ref[...].astype(jnp.float32)
    a = jnp.maximum(mid * sc_ref[:, 0:1] + sh_ref[:, 0:1], 0.0)
    r3 = a[:red3]
    r5 = a[red3:]

    wv3 = {dw: _wshift(r3, dw, w, 0.0) for dw in (-1, 0, 1)}
    taps3 = [_shift_lanes(wv3[kw - 1], (kh - 1) * w, 0.0)
             for kh in range(3) for kw in range(3)]
    x3 = jnp.concatenate(taps3, axis=0).astype(jnp.bfloat16)
    y3 = jnp.dot(w3_ref[...], x3, preferred_element_type=jnp.float32)

    wv5 = {dw: _wshift(r5, dw, w, 0.0) for dw in (-2, -1, 0, 1, 2)}
    taps5 = [_shift_lanes(wv5[kw - 2], (kh - 2) * w, 0.0)
             for kh in range(5) for kw in range(5)]
    x5 = jnp.concatenate(taps5, axis=0).astype(jnp.bfloat16)
    y5 = jnp.dot(w5_ref[...], x5, preferred_element_type=jnp.float32)

    out = jnp.concatenate([y3, y5], axis=0)
    acc2_ref[...] = out.astype(jnp.bfloat16)
    _accum_stats(st_ref, out, pl.program_id(0))


def _s3_body(keep_ref, acc2_ref, sc_ref, sh_ref, out_ref, *, out1):
    k = keep_ref[...]
    raw = jnp.concatenate([k[:out1], acc2_ref[...], k[out1:]],
                          axis=0).astype(jnp.float32)
    out_ref[...] = jnp.maximum(raw * sc_ref[:, 0:1] + sh_ref[:, 0:1], 0.0)


def _fold_bn(stats, gamma, beta, count):
    mean = stats[:, 0] / count
    var = stats[:, 1] / count - mean * mean
    scale = gamma * lax.rsqrt(var + _EPS)
    shift = beta - mean * scale
    return scale, shift


def _bcast(v):
    return jnp.broadcast_to(v.reshape(-1, 1), (v.shape[0], 128))


def kernel(x, b1_w, b1_gamma, b1_beta, b2a_w, b2a_gamma, b2a_beta,
           b2b_w, b2b_gamma, b2b_beta, b3a_w, b3a_gamma, b3a_beta,
           b3b_w, b3b_gamma, b3b_beta, b4_w, b4_gamma, b4_beta):
    x = x.astype(jnp.float32)
    n, cin, h, w = x.shape
    hw = h * w
    out1 = b1_w.shape[-1]
    red3, out3 = b2a_w.shape[-1], b2b_w.shape[-1]
    red5, out5 = b3a_w.shape[-1], b3b_w.shape[-1]
    outp = b4_w.shape[-1]
    cmid = red3 + red5
    ckeep = out1 + outp
    c2 = out3 + out5
    cout = out1 + out3 + out5 + outp

    x3 = x.reshape(n, cin, hw)
    wx = jnp.concatenate([b1_w.reshape(cin, out1),
                          b2a_w.reshape(cin, red3),
                          b3a_w.reshape(cin, red5)], axis=1)
    wx_t = wx.T.astype(jnp.bfloat16)
    wp_t = b4_w.reshape(cin, outp).T.astype(jnp.bfloat16)

    keep, mid, stats1 = pl.pallas_call(
        functools.partial(_s1_body, w=w, out1=out1),
        grid=(n,),
        in_specs=[
            pl.BlockSpec((None, cin, hw), lambda i: (i, 0, 0)),
            pl.BlockSpec((out1 + cmid, cin), lambda i: (0, 0)),
            pl.BlockSpec((outp, cin), lambda i: (0, 0)),
        ],
        out_specs=(
            pl.BlockSpec((None, ckeep, hw), lambda i: (i, 0, 0)),
            pl.BlockSpec((None, cmid, hw), lambda i: (i, 0, 0)),
            pl.BlockSpec((ckeep + cmid, 128), lambda i: (0, 0)),
        ),
        out_shape=(
            jax.ShapeDtypeStruct((n, ckeep, hw), jnp.bfloat16),
            jax.ShapeDtypeStruct((n, cmid, hw), jnp.bfloat16),
            jax.ShapeDtypeStruct((ckeep + cmid, 128), jnp.float32),
        ),
        compiler_params=pltpu.CompilerParams(
            dimension_semantics=("arbitrary",)),
    )(x3, wx_t, wp_t)

    count = float(n * hw)
    g_keep = jnp.concatenate([b1_gamma, b4_gamma])
    bt_keep = jnp.concatenate([b1_beta, b4_beta])
    g_mid = jnp.concatenate([b2a_gamma, b3a_gamma])
    bt_mid = jnp.concatenate([b2a_beta, b3a_beta])
    sc_k, sh_k = _fold_bn(stats1[:ckeep], g_keep, bt_keep, count)
    sc_m, sh_m = _fold_bn(stats1[ckeep:], g_mid, bt_mid, count)

    w3_t = b2b_w.reshape(9 * red3, out3).T.astype(jnp.bfloat16)
    w5_t = b3b_w.reshape(25 * red5, out5).T.astype(jnp.bfloat16)

    acc2, stats2 = pl.pallas_call(
        functools.partial(_s2_body, w=w, red3=red3),
        grid=(n,),
        in_specs=[
            pl.BlockSpec((None, cmid, hw), lambda i: (i, 0, 0)),
            pl.BlockSpec((cmid, 128), lambda i: (0, 0)),
            pl.BlockSpec((cmid, 128), lambda i: (0, 0)),
            pl.BlockSpec((out3, 9 * red3), lambda i: (0, 0)),
            pl.BlockSpec((out5, 25 * red5), lambda i: (0, 0)),
        ],
        out_specs=(
            pl.BlockSpec((None, c2, hw), lambda i: (i, 0, 0)),
            pl.BlockSpec((c2, 128), lambda i: (0, 0)),
        ),
        out_shape=(
            jax.ShapeDtypeStruct((n, c2, hw), jnp.bfloat16),
            jax.ShapeDtypeStruct((c2, 128), jnp.float32),
        ),
        compiler_params=pltpu.CompilerParams(
            dimension_semantics=("arbitrary",)),
    )(mid, _bcast(sc_m), _bcast(sh_m), w3_t, w5_t)

    g2 = jnp.concatenate([b2b_gamma, b3b_gamma])
    bt2 = jnp.concatenate([b2b_beta, b3b_beta])
    sc_2, sh_2 = _fold_bn(stats2, g2, bt2, count)

    sc_all = jnp.concatenate([sc_k[:out1], sc_2, sc_k[out1:]])
    sh_all = jnp.concatenate([sh_k[:out1], sh_2, sh_k[out1:]])

    out = pl.pallas_call(
        functools.partial(_s3_body, out1=out1),
        grid=(n,),
        in_specs=[
            pl.BlockSpec((None, ckeep, hw), lambda i: (i, 0, 0)),
            pl.BlockSpec((None, c2, hw), lambda i: (i, 0, 0)),
            pl.BlockSpec((cout, 128), lambda i: (0, 0)),
            pl.BlockSpec((cout, 128), lambda i: (0, 0)),
        ],
        out_specs=pl.BlockSpec((None, cout, hw), lambda i: (i, 0, 0)),
        out_shape=jax.ShapeDtypeStruct((n, cout, hw), jnp.float32),
        compiler_params=pltpu.CompilerParams(
            dimension_semantics=("parallel",)),
    )(keep, acc2, _bcast(sc_all), _bcast(sh_all))

    return out.reshape(n, cout, h, w)
